# baseline (speedup 1.0000x reference)
import functools

import jax
import jax.numpy as jnp
from jax import lax
from jax.experimental import pallas as pl
from jax.experimental.pallas import tpu as pltpu
from jax.experimental.pallas import tpu_sc as plsc

N_NODES = 10000
N_EDGES = 320000
C_IN = 128
NW = 32
EPT = N_EDGES // NW
K = 32
NCHUNK = 312
KT = 16
ZROWS = 80
NZ = N_NODES // ZROWS
CSTG = 80
CPAD = 10240


def _tables_body(x_ref, w1_ref, w2_ref, bnn_ref, a_ref, b_ref):
    x1 = x_ref[:, :C_IN]
    w2 = w2_ref[...]
    a_ref[...] = (
        jnp.dot(x1, w1_ref[...] - w2, preferred_element_type=jnp.float32)
        + bnn_ref[...]
    )
    b_ref[...] = jnp.dot(x1, w2, preferred_element_type=jnp.float32)


def _cmat_body(attr_ref, w3_ref, c_ref):
    a = attr_ref[...]
    w3 = w3_ref[...]
    acc = a[:, 0:1] * w3[0:1, :]
    for t in range(1, 4):
        acc = acc + a[:, t : t + 1] * w3[t : t + 1, :]
    c_ref[...] = acc


def _sc_edges_body(a_hbm, b_hbm, attr_hbm, w3_hbm, dst_hbm, src_hbm, w_hbm,
                   z_hbm,
                   out_hbm, cnt_hbm,
                   d0, d1, d2, s0, s1, s2, w0, w1, w2,
                   ga0, ga1, ga2, gb0, gb1, gb2, ab0, ab1, ab2,
                   w3v, didx_t, sidx_t, cnt_t, acc,
                   sem_i, sem_g, sem_s):
    cid = lax.axis_index("c")
    sid = lax.axis_index("s")
    wid = sid * 2 + cid
    didx = (d0, d1, d2)
    sidx = (s0, s1, s2)
    wbuf = (w0, w1, w2)
    ga = (ga0, ga1, ga2)
    gb = (gb0, gb1, gb2)
    abuf = (ab0, ab1, ab2)

    zv = jnp.zeros((16,), jnp.float32)
    ov = jnp.ones((16,), jnp.float32)

    for t in range((NZ + 15) // 16):
        c = sid + 16 * t

        @pl.when(c < NZ)
        def _zero_chunk():
            row0 = pl.multiple_of(c * ZROWS, 8)
            pltpu.sync_copy(z_hbm, acc.at[pl.ds(row0, ZROWS)])

    def _zcnt(r, carry):
        cnt_t[pl.ds(16 * r, 16)] = zv
        return carry

    lax.fori_loop(0, CPAD // 16, _zcnt, 0)

    pltpu.sync_copy(w3_hbm, w3v)

    plsc.subcore_barrier()

    ebase = wid * EPT

    def _issue_idx(k, m):
        e0 = ebase + k * K
        pltpu.async_copy(dst_hbm.at[pl.ds(e0, K)], didx[m], sem_i)
        pltpu.async_copy(src_hbm.at[pl.ds(e0, K)], sidx[m], sem_i)
        pltpu.async_copy(w_hbm.at[pl.ds(e0, K)], wbuf[m].at[pl.ds(0, K)], sem_i)

    def _wait_idx(m):
        pltpu.make_async_copy(dst_hbm.at[pl.ds(0, K)], didx[m], sem_i).wait()
        pltpu.make_async_copy(src_hbm.at[pl.ds(0, K)], sidx[m], sem_i).wait()
        pltpu.make_async_copy(
            w_hbm.at[pl.ds(0, K)], wbuf[m].at[pl.ds(0, K)], sem_i).wait()

    def _issue_gath(k, m):
        e0 = ebase + k * K
        pltpu.async_copy(a_hbm.at[didx[m]], ga[m], sem_g)
        pltpu.async_copy(b_hbm.at[sidx[m]], gb[m], sem_g)
        pltpu.async_copy(attr_hbm.at[pl.ds(4 * e0, 4 * K)], abuf[m], sem_g)

    def _wait_gath(m):
        pltpu.make_async_copy(a_hbm.at[didx[m]], ga[m], sem_g).wait()
        pltpu.make_async_copy(b_hbm.at[sidx[m]], gb[m], sem_g).wait()
        pltpu.make_async_copy(
            attr_hbm.at[pl.ds(0, 4 * K)], abuf[m], sem_g).wait()

    def _compute(m):
        def _hgrp(q, carry2):
            iv = didx[m][pl.ds(16 * q, 16)]
            plsc.addupdate_scatter(cnt_t, [iv], ov)
            return carry2

        lax.fori_loop(0, K // 16, _hgrp, 0)

        for j in range(C_IN // 16):
            sl = pl.ds(16 * j, 16)
            w3_0 = w3v[pl.ds(16 * j, 16)]
            w3_1 = w3v[pl.ds(C_IN + 16 * j, 16)]
            w3_2 = w3v[pl.ds(2 * C_IN + 16 * j, 16)]
            w3_3 = w3v[pl.ds(3 * C_IN + 16 * j, 16)]

            def _cj(q, carry2, sl=sl, w3_0=w3_0, w3_1=w3_1, w3_2=w3_2,
                    w3_3=w3_3):
                av = abuf[m][pl.ds(16 * q, 16)]
                wv = wbuf[m][pl.ds(4 * q, 16)]
                for i in range(4):
                    e = 4 * q + i
                    pre = (ga[m][e, sl] + gb[m][e, sl]
                           + av[4 * i] * w3_0 + av[4 * i + 1] * w3_1
                           + av[4 * i + 2] * w3_2 + av[4 * i + 3] * w3_3)
                    ga[m][e, sl] = jnp.maximum(pre, 0.0) * wv[i]
                return carry2

            lax.fori_loop(0, K // 4, _cj, 0)

    _issue_idx(0, 0)
    _wait_idx(0)
    _issue_gath(0, 0)
    _issue_idx(1, 1)

    NRING = NCHUNK // 3

    def _ring(kk, carry):
        for m in range(3):
            k = 3 * kk + m
            _wait_gath(m)
            if m == 0:
                @pl.when(kk > 0)
                def _ws0():
                    pltpu.make_async_copy(
                        ga[2], acc.at[didx[2]], sem_s).wait()
            else:
                pltpu.make_async_copy(
                    ga[m - 1], acc.at[didx[m - 1]], sem_s).wait()
            if m == 2:
                @pl.when(kk < NRING - 1)
                def _wi2():
                    _wait_idx(0)
                    _issue_gath(k + 1, 0)
            else:
                _wait_idx(m + 1)
                _issue_gath(k + 1, m + 1)
            _compute(m)
            pltpu.async_copy(ga[m], acc.at[didx[m]], sem_s, add=True)
            if m == 2:
                @pl.when(kk < NRING - 1)
                def _ii2():
                    _issue_idx(k + 2, 1)
            elif m == 1:
                @pl.when(kk < NRING - 1)
                def _ii1():
                    _issue_idx(k + 2, 0)
            else:
                _issue_idx(k + 2, 2)
        return carry

    lax.fori_loop(0, NRING, _ring, 0)

    pltpu.make_async_copy(ga[2], acc.at[didx[2]], sem_s).wait()

    e0t = ebase + NCHUNK * K
    pltpu.sync_copy(dst_hbm.at[pl.ds(e0t, KT)], didx_t)
    pltpu.sync_copy(src_hbm.at[pl.ds(e0t, KT)], sidx_t)
    pltpu.sync_copy(w_hbm.at[pl.ds(e0t, KT)], w1.at[pl.ds(0, KT)])
    pltpu.sync_copy(a_hbm.at[didx_t], ga1.at[pl.ds(0, KT)])
    pltpu.sync_copy(b_hbm.at[sidx_t], gb1.at[pl.ds(0, KT)])
    pltpu.sync_copy(attr_hbm.at[pl.ds(4 * e0t, 4 * KT)],
                    ab1.at[pl.ds(0, 4 * KT)])
    iv_t = didx_t[pl.ds(0, 16)]
    plsc.addupdate_scatter(cnt_t, [iv_t], ov)

    for j in range(C_IN // 16):
        sl = pl.ds(16 * j, 16)
        w3_0 = w3v[pl.ds(16 * j, 16)]
        w3_1 = w3v[pl.ds(C_IN + 16 * j, 16)]
        w3_2 = w3v[pl.ds(2 * C_IN + 16 * j, 16)]
        w3_3 = w3v[pl.ds(3 * C_IN + 16 * j, 16)]

        def _tj(q, carry2, sl=sl, w3_0=w3_0, w3_1=w3_1, w3_2=w3_2, w3_3=w3_3):
            av = ab1[pl.ds(16 * q, 16)]
            wv = w1[pl.ds(4 * q, 16)]
            for i in range(4):
                e = 4 * q + i
                pre = (ga1[e, sl] + gb1[e, sl]
                       + av[4 * i] * w3_0 + av[4 * i + 1] * w3_1
                       + av[4 * i + 2] * w3_2 + av[4 * i + 3] * w3_3)
                ga1[e, sl] = jnp.maximum(pre, 0.0) * wv[i]
            return carry2

        lax.fori_loop(0, KT // 4, _tj, 0)

    pltpu.sync_copy(ga1.at[pl.ds(0, KT)], acc.at[didx_t], add=True)

    plsc.subcore_barrier()

    for t in range((NZ + 15) // 16):
        c = sid + 16 * t

        @pl.when(c < NZ)
        def _dump_chunk():
            row0 = pl.multiple_of(c * ZROWS, 8)
            pltpu.sync_copy(acc.at[pl.ds(row0, ZROWS)],
                            out_hbm.at[cid, pl.ds(row0, ZROWS)])

    for d in range(3):
        rows = K if d < 2 else CSTG - 2 * K

        def _ccp(r, carry, d=d):
            for j in range(8):
                ga0[r, pl.ds(16 * j, 16)] = cnt_t[
                    pl.ds(d * K * 128 + r * 128 + 16 * j, 16)]
            return carry

        lax.fori_loop(0, rows, _ccp, 0)
        if rows == K:
            pltpu.sync_copy(ga0, cnt_hbm.at[wid, pl.ds(d * K, K)])
        else:
            pltpu.sync_copy(ga0.at[pl.ds(0, rows)],
                            cnt_hbm.at[wid, pl.ds(d * K, rows)])


def _combine_body(p_ref, cnt_ref, x_ref, wr_ref, br_ref, wm_ref, bm_ref, o_ref):
    s = p_ref[0] + p_ref[1]
    cnt = jnp.sum(cnt_ref[...], axis=1, keepdims=True)
    m = s / jnp.maximum(cnt, 1.0)
    x1 = x_ref[:, :C_IN]
    x2 = x_ref[:, C_IN:]
    m = m + jnp.dot(x1, wr_ref[...], preferred_element_type=jnp.float32) + br_ref[...]
    m = m + jnp.maximum(
        jnp.dot(x2, wm_ref[...], preferred_element_type=jnp.float32) + bm_ref[...],
        0.0,
    )
    o_ref[...] = m


def kernel(x, edge_index, edge_weight, edge_attr, W_nn, b_nn, W_r, b_r, W_m, b_m):
    f32 = jnp.float32
    x = x.astype(f32)
    w1 = W_nn[:C_IN].astype(f32)
    w2 = W_nn[C_IN : 2 * C_IN].astype(f32)
    w3 = W_nn[2 * C_IN :].astype(f32)
    bnn = b_nn.reshape(1, C_IN).astype(f32)
    br = b_r.reshape(1, C_IN).astype(f32)
    bm = b_m.reshape(1, C_IN).astype(f32)
    src = edge_index[0].astype(jnp.int32)
    dst = edge_index[1].astype(jnp.int32)
    ew = edge_weight.astype(f32)
    attr = edge_attr.astype(f32)
    zeros = jnp.zeros((ZROWS, C_IN), f32)

    R = 2000
    tables = pl.pallas_call(
        _tables_body,
        grid=(N_NODES // R,),
        in_specs=[
            pl.BlockSpec((R, 2 * C_IN), lambda i: (i, 0)),
            pl.BlockSpec((C_IN, C_IN), lambda i: (0, 0)),
            pl.BlockSpec((C_IN, C_IN), lambda i: (0, 0)),
            pl.BlockSpec((1, C_IN), lambda i: (0, 0)),
        ],
        out_specs=[
            pl.BlockSpec((R, C_IN), lambda i: (i, 0)),
            pl.BlockSpec((R, C_IN), lambda i: (i, 0)),
        ],
        out_shape=[
            jax.ShapeDtypeStruct((N_NODES, C_IN), f32),
            jax.ShapeDtypeStruct((N_NODES, C_IN), f32),
        ],
    )
    A, B = tables(x, w1, w2, bnn)

    attr_flat = attr.reshape(-1)
    w3_flat = w3.reshape(-1)
    sc_edges = functools.partial(
        pl.kernel,
        out_type=(
            jax.ShapeDtypeStruct((2, N_NODES, C_IN), f32),
            jax.ShapeDtypeStruct((NW, CSTG, C_IN), f32),
        ),
        mesh=plsc.VectorSubcoreMesh(core_axis_name="c", subcore_axis_name="s"),
        compiler_params=pltpu.CompilerParams(needs_layout_passes=False),
        scratch_types=(
            [pltpu.VMEM((K,), jnp.int32) for _ in range(3)]
            + [pltpu.VMEM((K,), jnp.int32) for _ in range(3)]
            + [pltpu.VMEM((K + 16,), f32) for _ in range(3)]
            + [pltpu.VMEM((K, C_IN), f32) for _ in range(3)]
            + [pltpu.VMEM((K, C_IN), f32) for _ in range(3)]
            + [pltpu.VMEM((4 * K,), f32) for _ in range(3)]
            + [
                pltpu.VMEM((4 * C_IN,), f32),
                pltpu.VMEM((KT,), jnp.int32),
                pltpu.VMEM((KT,), jnp.int32),
                pltpu.VMEM((CPAD,), f32),
                pltpu.VMEM_SHARED((N_NODES, C_IN), f32),
                pltpu.SemaphoreType.DMA,
                pltpu.SemaphoreType.DMA,
                pltpu.SemaphoreType.DMA,
            ]
        ),
    )(_sc_edges_body)
    P, CNT = sc_edges(A, B, attr_flat, w3_flat, dst, src, ew, zeros)
    CNTT = CNT.reshape(NW, CPAD)[:, :N_NODES].T

    combine = pl.pallas_call(
        _combine_body,
        grid=(N_NODES // R,),
        in_specs=[
            pl.BlockSpec((2, R, C_IN), lambda i: (0, i, 0)),
            pl.BlockSpec((R, NW), lambda i: (i, 0)),
            pl.BlockSpec((R, 2 * C_IN), lambda i: (i, 0)),
            pl.BlockSpec((C_IN, C_IN), lambda i: (0, 0)),
            pl.BlockSpec((1, C_IN), lambda i: (0, 0)),
            pl.BlockSpec((C_IN, C_IN), lambda i: (0, 0)),
            pl.BlockSpec((1, C_IN), lambda i: (0, 0)),
        ],
        out_specs=pl.BlockSpec((R, C_IN), lambda i: (i, 0)),
        out_shape=jax.ShapeDtypeStruct((N_NODES, C_IN), f32),
    )
    return combine(P, CNTT, x, W_r.astype(f32), br, W_m.astype(f32), bm)


# speedup vs baseline: 1.1935x; 1.1935x over previous
import functools

import jax
import jax.numpy as jnp
from jax import lax
from jax.experimental import pallas as pl
from jax.experimental.pallas import tpu as pltpu
from jax.experimental.pallas import tpu_sc as plsc

N_NODES = 10000
N_EDGES = 320000
C_IN = 128
NW = 32
EPT = N_EDGES // NW
K = 32
NCHUNK = 312
KT = 16
ZROWS = 80
NZ = N_NODES // ZROWS
CSTG = 80
CPAD = 10240


def _tables_body(x_ref, w1_ref, w2_ref, bnn_ref, a_ref, b_ref):
    x1 = x_ref[:, :C_IN]
    w2 = w2_ref[...]
    a_ref[...] = (
        jnp.dot(x1, w1_ref[...] - w2, preferred_element_type=jnp.float32)
        + bnn_ref[...]
    )
    b_ref[...] = jnp.dot(x1, w2, preferred_element_type=jnp.float32)


def _cmat_body(attr_ref, w3_ref, c_ref):
    a = attr_ref[...]
    w3 = w3_ref[...]
    acc = a[:, 0:1] * w3[0:1, :]
    for t in range(1, 4):
        acc = acc + a[:, t : t + 1] * w3[t : t + 1, :]
    c_ref[...] = acc


def _sc_edges_body(a_hbm, b_hbm, c_hbm, dst_hbm, src_hbm, w_hbm, z_hbm,
                   out_hbm, cnt_hbm,
                   d0, d1, d2, s0, s1, s2, w0, w1, w2,
                   ga0, ga1, ga2, gb0, gb1, gb2, gc0, gc1, gc2,
                   didx_t, sidx_t, cnt_t, acc,
                   sem_i, sem_g, sem_s):
    cid = lax.axis_index("c")
    sid = lax.axis_index("s")
    wid = sid * 2 + cid
    didx = (d0, d1, d2)
    sidx = (s0, s1, s2)
    wbuf = (w0, w1, w2)
    ga = (ga0, ga1, ga2)
    gb = (gb0, gb1, gb2)
    gc = (gc0, gc1, gc2)

    zv = jnp.zeros((16,), jnp.float32)
    ov = jnp.ones((16,), jnp.float32)

    for t in range((NZ + 15) // 16):
        c = sid + 16 * t

        @pl.when(c < NZ)
        def _zero_chunk():
            row0 = pl.multiple_of(c * ZROWS, 8)
            pltpu.sync_copy(z_hbm, acc.at[pl.ds(row0, ZROWS)])

    def _zcnt(r, carry):
        cnt_t[pl.ds(16 * r, 16)] = zv
        return carry

    lax.fori_loop(0, CPAD // 16, _zcnt, 0)

    plsc.subcore_barrier()

    ebase = wid * EPT

    def _issue_idx(k, m):
        e0 = ebase + k * K
        pltpu.async_copy(dst_hbm.at[pl.ds(e0, K)], didx[m], sem_i)
        pltpu.async_copy(src_hbm.at[pl.ds(e0, K)], sidx[m], sem_i)
        pltpu.async_copy(w_hbm.at[pl.ds(e0, K)], wbuf[m].at[pl.ds(0, K)], sem_i)

    def _wait_idx(m):
        pltpu.make_async_copy(dst_hbm.at[pl.ds(0, K)], didx[m], sem_i).wait()
        pltpu.make_async_copy(src_hbm.at[pl.ds(0, K)], sidx[m], sem_i).wait()
        pltpu.make_async_copy(
            w_hbm.at[pl.ds(0, K)], wbuf[m].at[pl.ds(0, K)], sem_i).wait()

    def _issue_gath(k, m):
        e0 = ebase + k * K
        pltpu.async_copy(a_hbm.at[didx[m]], ga[m], sem_g)
        pltpu.async_copy(b_hbm.at[sidx[m]], gb[m], sem_g)
        pltpu.async_copy(c_hbm.at[pl.ds(e0, K)], gc[m], sem_g)

    def _wait_gath(m):
        pltpu.make_async_copy(a_hbm.at[didx[m]], ga[m], sem_g).wait()
        pltpu.make_async_copy(b_hbm.at[sidx[m]], gb[m], sem_g).wait()
        pltpu.make_async_copy(c_hbm.at[pl.ds(0, K)], gc[m], sem_g).wait()

    def _compute(m):
        def _hgrp(q, carry2):
            iv = didx[m][pl.ds(16 * q, 16)]
            plsc.addupdate_scatter(cnt_t, [iv], ov)
            return carry2

        lax.fori_loop(0, K // 16, _hgrp, 0)

        def _e8(q, carry2):
            wv = wbuf[m][pl.ds(4 * q, 16)]
            for i in range(4):
                e = 4 * q + i
                w = wv[i]
                for j in range(C_IN // 16):
                    sl = pl.ds(16 * j, 16)
                    pre = ga[m][e, sl] + gb[m][e, sl] + gc[m][e, sl]
                    ga[m][e, sl] = jnp.maximum(pre, 0.0) * w
            return carry2

        lax.fori_loop(0, K // 4, _e8, 0)

    _issue_idx(0, 0)
    _wait_idx(0)
    _issue_gath(0, 0)
    _issue_idx(1, 1)

    NRING = NCHUNK // 3

    def _ring(kk, carry):
        for m in range(3):
            k = 3 * kk + m
            _wait_gath(m)
            if m == 0:
                @pl.when(kk > 0)
                def _ws0():
                    pltpu.make_async_copy(
                        ga[2], acc.at[didx[2]], sem_s).wait()
            else:
                pltpu.make_async_copy(
                    ga[m - 1], acc.at[didx[m - 1]], sem_s).wait()
            if m == 2:
                @pl.when(kk < NRING - 1)
                def _wi2():
                    _wait_idx(0)
                    _issue_gath(k + 1, 0)
            else:
                _wait_idx(m + 1)
                _issue_gath(k + 1, m + 1)
            _compute(m)
            pltpu.async_copy(ga[m], acc.at[didx[m]], sem_s, add=True)
            if m == 2:
                @pl.when(kk < NRING - 1)
                def _ii2():
                    _issue_idx(k + 2, 1)
            elif m == 1:
                @pl.when(kk < NRING - 1)
                def _ii1():
                    _issue_idx(k + 2, 0)
            else:
                _issue_idx(k + 2, 2)
        return carry

    lax.fori_loop(0, NRING, _ring, 0)

    pltpu.make_async_copy(ga[2], acc.at[didx[2]], sem_s).wait()

    e0t = ebase + NCHUNK * K
    pltpu.sync_copy(dst_hbm.at[pl.ds(e0t, KT)], didx_t)
    pltpu.sync_copy(src_hbm.at[pl.ds(e0t, KT)], sidx_t)
    pltpu.sync_copy(w_hbm.at[pl.ds(e0t, KT)], w1.at[pl.ds(0, KT)])
    pltpu.sync_copy(a_hbm.at[didx_t], ga1.at[pl.ds(0, KT)])
    pltpu.sync_copy(b_hbm.at[sidx_t], gb1.at[pl.ds(0, KT)])
    pltpu.sync_copy(c_hbm.at[pl.ds(e0t, KT)], gc1.at[pl.ds(0, KT)])
    iv_t = didx_t[pl.ds(0, 16)]
    plsc.addupdate_scatter(cnt_t, [iv_t], ov)

    def _t8(q, carry2):
        wv = w1[pl.ds(4 * q, 16)]
        for i in range(4):
            e = 4 * q + i
            w = wv[i]
            for j in range(C_IN // 16):
                sl = pl.ds(16 * j, 16)
                pre = ga1[e, sl] + gb1[e, sl] + gc1[e, sl]
                ga1[e, sl] = jnp.maximum(pre, 0.0) * w
        return carry2

    lax.fori_loop(0, KT // 4, _t8, 0)
    pltpu.sync_copy(ga1.at[pl.ds(0, KT)], acc.at[didx_t], add=True)

    plsc.subcore_barrier()

    for t in range((NZ + 15) // 16):
        c = sid + 16 * t

        @pl.when(c < NZ)
        def _dump_chunk():
            row0 = pl.multiple_of(c * ZROWS, 8)
            pltpu.sync_copy(acc.at[pl.ds(row0, ZROWS)],
                            out_hbm.at[cid, pl.ds(row0, ZROWS)])

    for d in range(3):
        rows = K if d < 2 else CSTG - 2 * K

        def _ccp(r, carry, d=d):
            for j in range(8):
                ga0[r, pl.ds(16 * j, 16)] = cnt_t[
                    pl.ds(d * K * 128 + r * 128 + 16 * j, 16)]
            return carry

        lax.fori_loop(0, rows, _ccp, 0)
        if rows == K:
            pltpu.sync_copy(ga0, cnt_hbm.at[wid, pl.ds(d * K, K)])
        else:
            pltpu.sync_copy(ga0.at[pl.ds(0, rows)],
                            cnt_hbm.at[wid, pl.ds(d * K, rows)])


def _combine_body(p_ref, cnt_ref, x_ref, wr_ref, br_ref, wm_ref, bm_ref, o_ref):
    s = p_ref[0] + p_ref[1]
    cnt = jnp.sum(cnt_ref[...], axis=1, keepdims=True)
    m = s / jnp.maximum(cnt, 1.0)
    x1 = x_ref[:, :C_IN]
    x2 = x_ref[:, C_IN:]
    m = m + jnp.dot(x1, wr_ref[...], preferred_element_type=jnp.float32) + br_ref[...]
    m = m + jnp.maximum(
        jnp.dot(x2, wm_ref[...], preferred_element_type=jnp.float32) + bm_ref[...],
        0.0,
    )
    o_ref[...] = m


def kernel(x, edge_index, edge_weight, edge_attr, W_nn, b_nn, W_r, b_r, W_m, b_m):
    f32 = jnp.float32
    x = x.astype(f32)
    w1 = W_nn[:C_IN].astype(f32)
    w2 = W_nn[C_IN : 2 * C_IN].astype(f32)
    w3 = W_nn[2 * C_IN :].astype(f32)
    bnn = b_nn.reshape(1, C_IN).astype(f32)
    br = b_r.reshape(1, C_IN).astype(f32)
    bm = b_m.reshape(1, C_IN).astype(f32)
    src = edge_index[0].astype(jnp.int32)
    dst = edge_index[1].astype(jnp.int32)
    ew = edge_weight.astype(f32)
    attr = edge_attr.astype(f32)
    zeros = jnp.zeros((ZROWS, C_IN), f32)

    R = 2000
    tables = pl.pallas_call(
        _tables_body,
        grid=(N_NODES // R,),
        in_specs=[
            pl.BlockSpec((R, 2 * C_IN), lambda i: (i, 0)),
            pl.BlockSpec((C_IN, C_IN), lambda i: (0, 0)),
            pl.BlockSpec((C_IN, C_IN), lambda i: (0, 0)),
            pl.BlockSpec((1, C_IN), lambda i: (0, 0)),
        ],
        out_specs=[
            pl.BlockSpec((R, C_IN), lambda i: (i, 0)),
            pl.BlockSpec((R, C_IN), lambda i: (i, 0)),
        ],
        out_shape=[
            jax.ShapeDtypeStruct((N_NODES, C_IN), f32),
            jax.ShapeDtypeStruct((N_NODES, C_IN), f32),
        ],
    )
    A, B = tables(x, w1, w2, bnn)

    EB = 8000
    cmat = pl.pallas_call(
        _cmat_body,
        grid=(N_EDGES // EB,),
        in_specs=[
            pl.BlockSpec((EB, 4), lambda i: (i, 0)),
            pl.BlockSpec((4, C_IN), lambda i: (0, 0)),
        ],
        out_specs=pl.BlockSpec((EB, C_IN), lambda i: (i, 0)),
        out_shape=jax.ShapeDtypeStruct((N_EDGES, C_IN), f32),
    )
    C = cmat(attr, w3)

    sc_edges = functools.partial(
        pl.kernel,
        out_type=(
            jax.ShapeDtypeStruct((2, N_NODES, C_IN), f32),
            jax.ShapeDtypeStruct((NW, CSTG, C_IN), f32),
        ),
        mesh=plsc.VectorSubcoreMesh(core_axis_name="c", subcore_axis_name="s"),
        compiler_params=pltpu.CompilerParams(needs_layout_passes=False),
        scratch_types=(
            [pltpu.VMEM((K,), jnp.int32) for _ in range(3)]
            + [pltpu.VMEM((K,), jnp.int32) for _ in range(3)]
            + [pltpu.VMEM((K + 16,), f32) for _ in range(3)]
            + [pltpu.VMEM((K, C_IN), f32) for _ in range(3)]
            + [pltpu.VMEM((K, C_IN), f32) for _ in range(3)]
            + [pltpu.VMEM((K, C_IN), f32) for _ in range(3)]
            + [
                pltpu.VMEM((KT,), jnp.int32),
                pltpu.VMEM((KT,), jnp.int32),
                pltpu.VMEM((CPAD,), f32),
                pltpu.VMEM_SHARED((N_NODES, C_IN), f32),
                pltpu.SemaphoreType.DMA,
                pltpu.SemaphoreType.DMA,
                pltpu.SemaphoreType.DMA,
            ]
        ),
    )(_sc_edges_body)
    P, CNT = sc_edges(A, B, C, dst, src, ew, zeros)
    CNTT = CNT.reshape(NW, CPAD)[:, :N_NODES].T

    combine = pl.pallas_call(
        _combine_body,
        grid=(N_NODES // R,),
        in_specs=[
            pl.BlockSpec((2, R, C_IN), lambda i: (0, i, 0)),
            pl.BlockSpec((R, NW), lambda i: (i, 0)),
            pl.BlockSpec((R, 2 * C_IN), lambda i: (i, 0)),
            pl.BlockSpec((C_IN, C_IN), lambda i: (0, 0)),
            pl.BlockSpec((1, C_IN), lambda i: (0, 0)),
            pl.BlockSpec((C_IN, C_IN), lambda i: (0, 0)),
            pl.BlockSpec((1, C_IN), lambda i: (0, 0)),
        ],
        out_specs=pl.BlockSpec((R, C_IN), lambda i: (i, 0)),
        out_shape=jax.ShapeDtypeStruct((N_NODES, C_IN), f32),
    )
    return combine(P, CNTT, x, W_r.astype(f32), br, W_m.astype(f32), bm)


# speedup vs baseline: 1.6192x; 1.3567x over previous
import functools

import jax
import jax.numpy as jnp
from jax import lax
from jax.experimental import pallas as pl
from jax.experimental.pallas import tpu as pltpu
from jax.experimental.pallas import tpu_sc as plsc

N_NODES = 10000
N_EDGES = 320000
C_IN = 128
NW = 32
EPT = N_EDGES // NW
K = 32
NCHUNK = 312
KT = 16
ZROWS = 80
NZ = N_NODES // ZROWS
CSTG = 80
CPAD = 10240


def _tables_body(x_ref, w1_ref, w2_ref, bnn_ref, a_ref, b_ref):
    x1 = x_ref[:, :C_IN]
    w2 = w2_ref[...]
    a_ref[...] = (
        jnp.dot(x1, w1_ref[...] - w2, preferred_element_type=jnp.float32)
        + bnn_ref[...]
    )
    b_ref[...] = jnp.dot(x1, w2, preferred_element_type=jnp.float32)


def _cmat_body(attr_ref, w3_ref, c_ref):
    at = attr_ref[...]
    w3 = w3_ref[...]
    c_ref[...] = lax.dot_general(
        at, w3, (((0,), (0,)), ((), ())),
        preferred_element_type=jnp.float32)


def _sc_edges_body(a_hbm, b_hbm, c_hbm, dst_hbm, src_hbm, w_hbm, z_hbm,
                   out_hbm, cnt_hbm,
                   d0, d1, d2, s0, s1, s2, w0, w1, w2,
                   ga0, ga1, ga2, gb0, gb1, gb2, gc0, gc1, gc2,
                   didx_t, sidx_t, cnt_t, acc,
                   sem_i, sem_g, sem_s):
    cid = lax.axis_index("c")
    sid = lax.axis_index("s")
    wid = sid * 2 + cid
    didx = (d0, d1, d2)
    sidx = (s0, s1, s2)
    wbuf = (w0, w1, w2)
    ga = (ga0, ga1, ga2)
    gb = (gb0, gb1, gb2)
    gc = (gc0, gc1, gc2)

    zv = jnp.zeros((16,), jnp.float32)
    ov = jnp.ones((16,), jnp.float32)

    for t in range((NZ + 15) // 16):
        c = sid + 16 * t

        @pl.when(c < NZ)
        def _zero_chunk():
            row0 = pl.multiple_of(c * ZROWS, 8)
            pltpu.sync_copy(z_hbm, acc.at[pl.ds(row0, ZROWS)])

    def _zcnt(r, carry):
        cnt_t[pl.ds(16 * r, 16)] = zv
        return carry

    lax.fori_loop(0, CPAD // 16, _zcnt, 0)

    plsc.subcore_barrier()

    ebase = wid * EPT

    def _issue_idx(k, m):
        e0 = ebase + k * K
        pltpu.async_copy(dst_hbm.at[pl.ds(e0, K)], didx[m], sem_i)
        pltpu.async_copy(src_hbm.at[pl.ds(e0, K)], sidx[m], sem_i)
        pltpu.async_copy(w_hbm.at[pl.ds(e0, K)], wbuf[m].at[pl.ds(0, K)], sem_i)

    def _wait_idx(m):
        pltpu.make_async_copy(dst_hbm.at[pl.ds(0, K)], didx[m], sem_i).wait()
        pltpu.make_async_copy(src_hbm.at[pl.ds(0, K)], sidx[m], sem_i).wait()
        pltpu.make_async_copy(
            w_hbm.at[pl.ds(0, K)], wbuf[m].at[pl.ds(0, K)], sem_i).wait()

    def _issue_gath(k, m):
        e0 = ebase + k * K
        pltpu.async_copy(a_hbm.at[didx[m]], ga[m], sem_g)
        pltpu.async_copy(b_hbm.at[sidx[m]], gb[m], sem_g)
        pltpu.async_copy(c_hbm.at[pl.ds(e0, K)], gc[m], sem_g)

    def _wait_gath(m):
        pltpu.make_async_copy(a_hbm.at[didx[m]], ga[m], sem_g).wait()
        pltpu.make_async_copy(b_hbm.at[sidx[m]], gb[m], sem_g).wait()
        pltpu.make_async_copy(c_hbm.at[pl.ds(0, K)], gc[m], sem_g).wait()

    def _compute(m):
        def _hgrp(q, carry2):
            iv = didx[m][pl.ds(16 * q, 16)]
            plsc.addupdate_scatter(cnt_t, [iv], ov)
            return carry2

        lax.fori_loop(0, K // 16, _hgrp, 0)

        def _e8(q, carry2):
            wv = wbuf[m][pl.ds(4 * q, 16)]
            for i in range(4):
                e = 4 * q + i
                w = wv[i]
                for j in range(C_IN // 16):
                    sl = pl.ds(16 * j, 16)
                    pre = ga[m][e, sl] + gb[m][e, sl] + gc[m][e, sl]
                    ga[m][e, sl] = jnp.maximum(pre, 0.0) * w
            return carry2

        lax.fori_loop(0, K // 4, _e8, 0)

    _issue_idx(0, 0)
    _wait_idx(0)
    _issue_gath(0, 0)
    _issue_idx(1, 1)

    NRING = NCHUNK // 3

    def _ring(kk, carry):
        for m in range(3):
            k = 3 * kk + m
            _wait_gath(m)
            if m == 0:
                @pl.when(kk > 0)
                def _ws0():
                    pltpu.make_async_copy(
                        ga[2], acc.at[didx[2]], sem_s).wait()
            else:
                pltpu.make_async_copy(
                    ga[m - 1], acc.at[didx[m - 1]], sem_s).wait()
            if m == 2:
                @pl.when(kk < NRING - 1)
                def _wi2():
                    _wait_idx(0)
                    _issue_gath(k + 1, 0)
            else:
                _wait_idx(m + 1)
                _issue_gath(k + 1, m + 1)
            _compute(m)
            pltpu.async_copy(ga[m], acc.at[didx[m]], sem_s, add=True)
            if m == 2:
                @pl.when(kk < NRING - 1)
                def _ii2():
                    _issue_idx(k + 2, 1)
            elif m == 1:
                @pl.when(kk < NRING - 1)
                def _ii1():
                    _issue_idx(k + 2, 0)
            else:
                _issue_idx(k + 2, 2)
        return carry

    lax.fori_loop(0, NRING, _ring, 0)

    pltpu.make_async_copy(ga[2], acc.at[didx[2]], sem_s).wait()

    e0t = ebase + NCHUNK * K
    pltpu.sync_copy(dst_hbm.at[pl.ds(e0t, KT)], didx_t)
    pltpu.sync_copy(src_hbm.at[pl.ds(e0t, KT)], sidx_t)
    pltpu.sync_copy(w_hbm.at[pl.ds(e0t, KT)], w1.at[pl.ds(0, KT)])
    pltpu.sync_copy(a_hbm.at[didx_t], ga1.at[pl.ds(0, KT)])
    pltpu.sync_copy(b_hbm.at[sidx_t], gb1.at[pl.ds(0, KT)])
    pltpu.sync_copy(c_hbm.at[pl.ds(e0t, KT)], gc1.at[pl.ds(0, KT)])
    iv_t = didx_t[pl.ds(0, 16)]
    plsc.addupdate_scatter(cnt_t, [iv_t], ov)

    def _t8(q, carry2):
        wv = w1[pl.ds(4 * q, 16)]
        for i in range(4):
            e = 4 * q + i
            w = wv[i]
            for j in range(C_IN // 16):
                sl = pl.ds(16 * j, 16)
                pre = ga1[e, sl] + gb1[e, sl] + gc1[e, sl]
                ga1[e, sl] = jnp.maximum(pre, 0.0) * w
        return carry2

    lax.fori_loop(0, KT // 4, _t8, 0)
    pltpu.sync_copy(ga1.at[pl.ds(0, KT)], acc.at[didx_t], add=True)

    plsc.subcore_barrier()

    for t in range((NZ + 15) // 16):
        c = sid + 16 * t

        @pl.when(c < NZ)
        def _dump_chunk():
            row0 = pl.multiple_of(c * ZROWS, 8)
            pltpu.sync_copy(acc.at[pl.ds(row0, ZROWS)],
                            out_hbm.at[cid, pl.ds(row0, ZROWS)])

    for d in range(3):
        rows = K if d < 2 else CSTG - 2 * K

        def _ccp(r, carry, d=d):
            for j in range(8):
                ga0[r, pl.ds(16 * j, 16)] = cnt_t[
                    pl.ds(d * K * 128 + r * 128 + 16 * j, 16)]
            return carry

        lax.fori_loop(0, rows, _ccp, 0)
        if rows == K:
            pltpu.sync_copy(ga0, cnt_hbm.at[wid, pl.ds(d * K, K)])
        else:
            pltpu.sync_copy(ga0.at[pl.ds(0, rows)],
                            cnt_hbm.at[wid, pl.ds(d * K, rows)])


def _combine_body(p_ref, cnt_ref, x_ref, wr_ref, br_ref, wm_ref, bm_ref, o_ref):
    s = p_ref[0] + p_ref[1]
    cnt = jnp.sum(cnt_ref[...], axis=1, keepdims=True)
    m = s / jnp.maximum(cnt, 1.0)
    x1 = x_ref[:, :C_IN]
    x2 = x_ref[:, C_IN:]
    m = m + jnp.dot(x1, wr_ref[...], preferred_element_type=jnp.float32) + br_ref[...]
    m = m + jnp.maximum(
        jnp.dot(x2, wm_ref[...], preferred_element_type=jnp.float32) + bm_ref[...],
        0.0,
    )
    o_ref[...] = m


def kernel(x, edge_index, edge_weight, edge_attr, W_nn, b_nn, W_r, b_r, W_m, b_m):
    f32 = jnp.float32
    x = x.astype(f32)
    w1 = W_nn[:C_IN].astype(f32)
    w2 = W_nn[C_IN : 2 * C_IN].astype(f32)
    w3 = W_nn[2 * C_IN :].astype(f32)
    bnn = b_nn.reshape(1, C_IN).astype(f32)
    br = b_r.reshape(1, C_IN).astype(f32)
    bm = b_m.reshape(1, C_IN).astype(f32)
    src = edge_index[0].astype(jnp.int32)
    dst = edge_index[1].astype(jnp.int32)
    ew = edge_weight.astype(f32)
    attr = edge_attr.astype(f32)
    zeros = jnp.zeros((ZROWS, C_IN), f32)

    R = 2000
    tables = pl.pallas_call(
        _tables_body,
        grid=(N_NODES // R,),
        in_specs=[
            pl.BlockSpec((R, 2 * C_IN), lambda i: (i, 0)),
            pl.BlockSpec((C_IN, C_IN), lambda i: (0, 0)),
            pl.BlockSpec((C_IN, C_IN), lambda i: (0, 0)),
            pl.BlockSpec((1, C_IN), lambda i: (0, 0)),
        ],
        out_specs=[
            pl.BlockSpec((R, C_IN), lambda i: (i, 0)),
            pl.BlockSpec((R, C_IN), lambda i: (i, 0)),
        ],
        out_shape=[
            jax.ShapeDtypeStruct((N_NODES, C_IN), f32),
            jax.ShapeDtypeStruct((N_NODES, C_IN), f32),
        ],
    )
    A, B = tables(x, w1, w2, bnn)

    EB = 16000
    cmat = pl.pallas_call(
        _cmat_body,
        grid=(N_EDGES // EB,),
        in_specs=[
            pl.BlockSpec((4, EB), lambda i: (0, i)),
            pl.BlockSpec((4, C_IN), lambda i: (0, 0)),
        ],
        out_specs=pl.BlockSpec((EB, C_IN), lambda i: (i, 0)),
        out_shape=jax.ShapeDtypeStruct((N_EDGES, C_IN), f32),
    )
    C = cmat(attr.T, w3)

    sc_edges = functools.partial(
        pl.kernel,
        out_type=(
            jax.ShapeDtypeStruct((2, N_NODES, C_IN), f32),
            jax.ShapeDtypeStruct((NW, CSTG, C_IN), f32),
        ),
        mesh=plsc.VectorSubcoreMesh(core_axis_name="c", subcore_axis_name="s"),
        compiler_params=pltpu.CompilerParams(needs_layout_passes=False),
        scratch_types=(
            [pltpu.VMEM((K,), jnp.int32) for _ in range(3)]
            + [pltpu.VMEM((K,), jnp.int32) for _ in range(3)]
            + [pltpu.VMEM((K + 16,), f32) for _ in range(3)]
            + [pltpu.VMEM((K, C_IN), f32) for _ in range(3)]
            + [pltpu.VMEM((K, C_IN), f32) for _ in range(3)]
            + [pltpu.VMEM((K, C_IN), f32) for _ in range(3)]
            + [
                pltpu.VMEM((KT,), jnp.int32),
                pltpu.VMEM((KT,), jnp.int32),
                pltpu.VMEM((CPAD,), f32),
                pltpu.VMEM_SHARED((N_NODES, C_IN), f32),
                pltpu.SemaphoreType.DMA,
                pltpu.SemaphoreType.DMA,
                pltpu.SemaphoreType.DMA,
            ]
        ),
    )(_sc_edges_body)
    P, CNT = sc_edges(A, B, C, dst, src, ew, zeros)
    CNTT = CNT.reshape(NW, CPAD)[:, :N_NODES].T

    combine = pl.pallas_call(
        _combine_body,
        grid=(N_NODES // R,),
        in_specs=[
            pl.BlockSpec((2, R, C_IN), lambda i: (0, i, 0)),
            pl.BlockSpec((R, NW), lambda i: (i, 0)),
            pl.BlockSpec((R, 2 * C_IN), lambda i: (i, 0)),
            pl.BlockSpec((C_IN, C_IN), lambda i: (0, 0)),
            pl.BlockSpec((1, C_IN), lambda i: (0, 0)),
            pl.BlockSpec((C_IN, C_IN), lambda i: (0, 0)),
            pl.BlockSpec((1, C_IN), lambda i: (0, 0)),
        ],
        out_specs=pl.BlockSpec((R, C_IN), lambda i: (i, 0)),
        out_shape=jax.ShapeDtypeStruct((N_NODES, C_IN), f32),
    )
    return combine(P, CNTT, x, W_r.astype(f32), br, W_m.astype(f32), bm)
